# baseline (speedup 1.0000x reference)
import functools

import jax
import jax.numpy as jnp
from jax import lax
from jax.experimental import pallas as pl
from jax.experimental.pallas import tpu as pltpu


def _rb(n):
    for cand in (2000, 1000, 500, 8):
        if n % cand == 0:
            return cand
    return n


def _chunk(n, d):
    budget = 24 * 1024 * 1024 // (4 * d)
    if n <= budget:
        return n
    for cand in (16000, 8000, 4000, 2000, 1000):
        if n % cand == 0 and cand <= budget:
            return cand
    return _rb(n)


def _ln_silu(h, g, be):
    mu = jnp.mean(h, axis=-1, keepdims=True)
    var = jnp.mean((h - mu) ** 2, axis=-1, keepdims=True)
    h = (h - mu) / jnp.sqrt(var + 1e-5) * g + be
    return h * jax.nn.sigmoid(h)


def _dense_body(x_ref, w_ref, b_ref, g_ref, be_ref, o_ref, *, ln):
    h = jnp.dot(x_ref[...], w_ref[...], preferred_element_type=jnp.float32)
    h = h + b_ref[...]
    if ln:
        h = _ln_silu(h, g_ref[...], be_ref[...])
    o_ref[...] = h


def _dense(x, w, b, g=None, be=None):
    n, din = x.shape
    dout = w.shape[1]
    rb = _rb(n)
    ln = g is not None
    if g is None:
        g = jnp.ones((dout,), jnp.float32)
        be = jnp.zeros((dout,), jnp.float32)
    return pl.pallas_call(
        functools.partial(_dense_body, ln=ln),
        grid=(n // rb,),
        in_specs=[
            pl.BlockSpec((rb, din), lambda i: (i, 0)),
            pl.BlockSpec((din, dout), lambda i: (0, 0)),
            pl.BlockSpec((1, dout), lambda i: (0, 0)),
            pl.BlockSpec((1, dout), lambda i: (0, 0)),
            pl.BlockSpec((1, dout), lambda i: (0, 0)),
        ],
        out_specs=pl.BlockSpec((rb, dout), lambda i: (i, 0)),
        out_shape=jax.ShapeDtypeStruct((n, dout), jnp.float32),
    )(x, w, b.reshape(1, dout), g.reshape(1, dout), be.reshape(1, dout))


def _embed_body(v_ref, w1_ref, b1_ref, g1_ref, be1_ref,
                w2_ref, b2_ref, g2_ref, be2_ref, o_ref,
                *, vmin, vmax, bins, norm):
    v = v_ref[...]
    if norm:
        d = jnp.sqrt(jnp.sum(v * v, axis=1, keepdims=True))
    else:
        d = v[:, 0:1]
    step = (vmax - vmin) / (bins - 1)
    centers = (lax.broadcasted_iota(jnp.int32, (1, bins), 1).astype(jnp.float32)
               * step + vmin)
    gamma = 1.0 / ((vmax - vmin) / (bins - 1))
    rbf = jnp.exp(-gamma * (d - centers) ** 2)
    h = jnp.dot(rbf, w1_ref[...], preferred_element_type=jnp.float32) + b1_ref[...]
    h = _ln_silu(h, g1_ref[...], be1_ref[...])
    h = jnp.dot(h, w2_ref[...], preferred_element_type=jnp.float32) + b2_ref[...]
    o_ref[...] = _ln_silu(h, g2_ref[...], be2_ref[...])


def _embed(v, p1, p2, vmin, vmax, bins, norm):
    n, c = v.shape
    rb = _rb(n)
    d1 = p1['W'].shape[1]
    d2 = p2['W'].shape[1]
    return pl.pallas_call(
        functools.partial(_embed_body, vmin=vmin, vmax=vmax, bins=bins, norm=norm),
        grid=(n // rb,),
        in_specs=[
            pl.BlockSpec((rb, c), lambda i: (i, 0)),
            pl.BlockSpec((bins, d1), lambda i: (0, 0)),
            pl.BlockSpec((1, d1), lambda i: (0, 0)),
            pl.BlockSpec((1, d1), lambda i: (0, 0)),
            pl.BlockSpec((1, d1), lambda i: (0, 0)),
            pl.BlockSpec((d1, d2), lambda i: (0, 0)),
            pl.BlockSpec((1, d2), lambda i: (0, 0)),
            pl.BlockSpec((1, d2), lambda i: (0, 0)),
            pl.BlockSpec((1, d2), lambda i: (0, 0)),
        ],
        out_specs=pl.BlockSpec((rb, d2), lambda i: (i, 0)),
        out_shape=jax.ShapeDtypeStruct((n, d2), jnp.float32),
    )(v, p1['W'], p1['b'].reshape(1, d1), p1['g'].reshape(1, d1),
      p1['be'].reshape(1, d1), p2['W'], p2['b'].reshape(1, d2),
      p2['g'].reshape(1, d2), p2['be'].reshape(1, d2))


def _gather_body(idx_ref, t_ref, o_ref, *, rb, ch):
    c = pl.program_id(1)
    lo = c * ch

    @pl.when(c == 0)
    def _():
        o_ref[...] = jnp.zeros_like(o_ref)

    def body(i, carry):
        ix = idx_ref[0, 0, i]
        li = ix - lo

        @pl.when(jnp.logical_and(ix >= lo, ix < lo + ch))
        def _():
            o_ref[pl.ds(i, 1), :] = t_ref[pl.ds(li, 1), :]

        return carry

    lax.fori_loop(0, rb, body, 0)


def _gather(table, idx):
    n, d = table.shape
    ne = idx.shape[0]
    rb = _rb(ne)
    ch = _chunk(n, d)
    idx3 = idx.reshape(ne // rb, 1, rb)
    return pl.pallas_call(
        functools.partial(_gather_body, rb=rb, ch=ch),
        grid=(ne // rb, n // ch),
        in_specs=[
            pl.BlockSpec((1, 1, rb), lambda e, c: (e, 0, 0),
                         memory_space=pltpu.SMEM),
            pl.BlockSpec((ch, d), lambda e, c: (c, 0)),
        ],
        out_specs=pl.BlockSpec((rb, d), lambda e, c: (e, 0)),
        out_shape=jax.ShapeDtypeStruct((ne, d), jnp.float32),
    )(idx3, table)


def _scatter_body(idx_ref, v_ref, o_ref, *, rb, ch):
    e = pl.program_id(1)
    lo = pl.program_id(0) * ch

    @pl.when(e == 0)
    def _():
        o_ref[...] = jnp.zeros_like(o_ref)

    def body(i, carry):
        ix = idx_ref[0, 0, i]
        li = ix - lo

        @pl.when(jnp.logical_and(ix >= lo, ix < lo + ch))
        def _():
            o_ref[pl.ds(li, 1), :] += v_ref[pl.ds(i, 1), :]

        return carry

    lax.fori_loop(0, rb, body, 0)


def _scatter_add(vals, idx, n):
    ne, d = vals.shape
    rb = _rb(ne)
    ch = _chunk(n, d)
    idx3 = idx.reshape(ne // rb, 1, rb)
    return pl.pallas_call(
        functools.partial(_scatter_body, rb=rb, ch=ch),
        grid=(n // ch, ne // rb),
        in_specs=[
            pl.BlockSpec((1, 1, rb), lambda c, e: (e, 0, 0),
                         memory_space=pltpu.SMEM),
            pl.BlockSpec((rb, d), lambda c, e: (e, 0)),
        ],
        out_specs=pl.BlockSpec((ch, d), lambda c, e: (c, 0)),
        out_shape=jax.ShapeDtypeStruct((n, d), jnp.float32),
    )(idx3, vals)


def _msig_body(gs_ref, gd_ref, ye_ref, m_ref, cs_ref, *, h):
    gs = gs_ref[...]
    xd = gd_ref[...]
    xs = gs[:, :h]
    bh = gs[:, h:]
    m = xs + xd + ye_ref[...]
    sig = jax.nn.sigmoid(m)
    m_ref[...] = m
    cs_ref[...] = jnp.concatenate([bh * sig, sig], axis=1)


def _msig(gs, gd, ye):
    ne, h = ye.shape
    rb = _rb(ne)
    return pl.pallas_call(
        functools.partial(_msig_body, h=h),
        grid=(ne // rb,),
        in_specs=[
            pl.BlockSpec((rb, 2 * h), lambda i: (i, 0)),
            pl.BlockSpec((rb, h), lambda i: (i, 0)),
            pl.BlockSpec((rb, h), lambda i: (i, 0)),
        ],
        out_specs=[
            pl.BlockSpec((rb, h), lambda i: (i, 0)),
            pl.BlockSpec((rb, 2 * h), lambda i: (i, 0)),
        ],
        out_shape=[
            jax.ShapeDtypeStruct((ne, h), jnp.float32),
            jax.ShapeDtypeStruct((ne, 2 * h), jnp.float32),
        ],
    )(gs, gd, ye)


def _xout_body(xsu_ref, nd_ref, x_ref, g_ref, be_ref, o_ref, *, h):
    nd = nd_ref[...]
    t = xsu_ref[...] + nd[:, :h] / (nd[:, h:] + 1e-6)
    o_ref[...] = x_ref[...] + _ln_silu(t, g_ref[...], be_ref[...])


def _xout(xsu, nd, x, g, be):
    n, h = x.shape
    rb = _rb(n)
    return pl.pallas_call(
        functools.partial(_xout_body, h=h),
        grid=(n // rb,),
        in_specs=[
            pl.BlockSpec((rb, h), lambda i: (i, 0)),
            pl.BlockSpec((rb, 2 * h), lambda i: (i, 0)),
            pl.BlockSpec((rb, h), lambda i: (i, 0)),
            pl.BlockSpec((1, h), lambda i: (0, 0)),
            pl.BlockSpec((1, h), lambda i: (0, 0)),
        ],
        out_specs=pl.BlockSpec((rb, h), lambda i: (i, 0)),
        out_shape=jax.ShapeDtypeStruct((n, h), jnp.float32),
    )(xsu, nd, x, g.reshape(1, h), be.reshape(1, h))


def _yout_body(m_ref, y_ref, g_ref, be_ref, o_ref):
    o_ref[...] = y_ref[...] + _ln_silu(m_ref[...], g_ref[...], be_ref[...])


def _yout(m, y, g, be):
    n, h = y.shape
    rb = _rb(n)
    return pl.pallas_call(
        _yout_body,
        grid=(n // rb,),
        in_specs=[
            pl.BlockSpec((rb, h), lambda i: (i, 0)),
            pl.BlockSpec((rb, h), lambda i: (i, 0)),
            pl.BlockSpec((1, h), lambda i: (0, 0)),
            pl.BlockSpec((1, h), lambda i: (0, 0)),
        ],
        out_specs=pl.BlockSpec((rb, h), lambda i: (i, 0)),
        out_shape=jax.ShapeDtypeStruct((n, h), jnp.float32),
    )(m, y, g.reshape(1, h), be.reshape(1, h))


def _pool_body(x_ref, fcw_ref, fcb_ref, o_ref, acc_ref, *, n):
    i = pl.program_id(0)

    @pl.when(i == 0)
    def _():
        acc_ref[...] = jnp.zeros_like(acc_ref)

    acc_ref[0:1, :] += jnp.sum(x_ref[...], axis=0, keepdims=True)

    @pl.when(i == pl.num_programs(0) - 1)
    def _():
        o_ref[0, 0] = jnp.sum(acc_ref[0:1, :] * fcw_ref[...]) / n + fcb_ref[0, 0]


def _pool_fc(x, fcw, fcb):
    n, h = x.shape
    rb = _rb(n)
    out = pl.pallas_call(
        functools.partial(_pool_body, n=n),
        grid=(n // rb,),
        in_specs=[
            pl.BlockSpec((rb, h), lambda i: (i, 0)),
            pl.BlockSpec((1, h), lambda i: (0, 0)),
            pl.BlockSpec((1, 1), lambda i: (0, 0), memory_space=pltpu.SMEM),
        ],
        out_specs=pl.BlockSpec(memory_space=pltpu.SMEM),
        out_shape=jax.ShapeDtypeStruct((1, 1), jnp.float32),
        scratch_shapes=[pltpu.VMEM((8, h), jnp.float32)],
    )(x, fcw.reshape(1, h), fcb.reshape(1, 1))
    return out[0, 0]


def _egc(src, dst, n, x, y, p):
    h = x.shape[1]
    wcat = jnp.concatenate([p['sgW'], p['duW'], p['dgW'], p['suW']], axis=1)
    bcat = jnp.concatenate([p['sgb'], p['dub'], p['dgb'], p['sub']], axis=0)
    xcat = _dense(x, wcat, bcat)
    ye = _dense(y, p['egW'], p['egb'])
    gs = _gather(xcat[:, :2 * h], src)
    gd = _gather(xcat[:, 2 * h:3 * h], dst)
    m, cs = _msig(gs, gd, ye)
    nd = _scatter_add(cs, dst, n)
    xo = _xout(xcat[:, 3 * h:], nd, x, p['lnNg'], p['lnNb'])
    yo = _yout(m, y, p['lnEg'], p['lnEb'])
    return xo, yo


def kernel(atom_features, r, angle_h, edge_index, lg_edge_index, params):
    n_nodes = atom_features.shape[0]
    n_edges = r.shape[0]
    src, dst = edge_index[0], edge_index[1]
    lsrc, ldst = lg_edge_index[0], lg_edge_index[1]

    af = jnp.pad(atom_features, ((0, 0), (0, 128 - atom_features.shape[1])))
    aw = jnp.pad(params['atom_emb']['W'],
                 ((0, 128 - params['atom_emb']['W'].shape[0]), (0, 0)))
    x = _dense(af, aw, params['atom_emb']['b'],
               params['atom_emb']['g'], params['atom_emb']['be'])

    r8 = jnp.pad(r, ((0, 0), (0, 8 - r.shape[1])))
    y = _embed(r8, params['edge_mlp1'], params['edge_mlp2'],
               0.0, 8.0, 80, norm=True)
    z = _embed(angle_h.reshape(-1, 1), params['ang_mlp1'], params['ang_mlp2'],
               -1.0, 1.0, 40, norm=False)

    for lp in params['alignn']:
        x, m = _egc(src, dst, n_nodes, x, y, lp['node'])
        y, z = _egc(lsrc, ldst, n_edges, m, z, lp['edge'])
    for gp in params['gcn']:
        x, y = _egc(src, dst, n_nodes, x, y, gp)

    return _pool_fc(x, params['fcW'], params['fcb'])


# speedup vs baseline: 1.2277x; 1.2277x over previous
import functools

import jax
import jax.numpy as jnp
from jax import lax
from jax.experimental import pallas as pl
from jax.experimental.pallas import tpu as pltpu


def _rb(n):
    for cand in (2000, 1000, 500, 8):
        if n % cand == 0:
            return cand
    return n


def _chunk(n, d):
    budget = 20 * 1024 * 1024 // (4 * d)
    if n <= budget:
        return n
    for cand in (20000, 16000, 10000, 8000, 4000, 2000, 1000):
        if n % cand == 0 and cand <= budget:
            return cand
    return _rb(n)


def _ln_silu(h, g, be):
    mu = jnp.mean(h, axis=-1, keepdims=True)
    var = jnp.mean((h - mu) ** 2, axis=-1, keepdims=True)
    h = (h - mu) / jnp.sqrt(var + 1e-5) * g + be
    return h * jax.nn.sigmoid(h)


def _dense_body(x_ref, w_ref, b_ref, g_ref, be_ref, o_ref, *, ln):
    h = jnp.dot(x_ref[...], w_ref[...], preferred_element_type=jnp.float32)
    h = h + b_ref[...]
    if ln:
        h = _ln_silu(h, g_ref[...], be_ref[...])
    o_ref[...] = h


def _dense(x, w, b, g=None, be=None):
    n, din = x.shape
    dout = w.shape[1]
    rb = _rb(n)
    ln = g is not None
    if g is None:
        g = jnp.ones((dout,), jnp.float32)
        be = jnp.zeros((dout,), jnp.float32)
    return pl.pallas_call(
        functools.partial(_dense_body, ln=ln),
        grid=(n // rb,),
        in_specs=[
            pl.BlockSpec((rb, din), lambda i: (i, 0)),
            pl.BlockSpec((din, dout), lambda i: (0, 0)),
            pl.BlockSpec((1, dout), lambda i: (0, 0)),
            pl.BlockSpec((1, dout), lambda i: (0, 0)),
            pl.BlockSpec((1, dout), lambda i: (0, 0)),
        ],
        out_specs=pl.BlockSpec((rb, dout), lambda i: (i, 0)),
        out_shape=jax.ShapeDtypeStruct((n, dout), jnp.float32),
    )(x, w, b.reshape(1, dout), g.reshape(1, dout), be.reshape(1, dout))


def _embed_body(v_ref, w1_ref, b1_ref, g1_ref, be1_ref,
                w2_ref, b2_ref, g2_ref, be2_ref, o_ref,
                *, vmin, vmax, bins, norm):
    v = v_ref[...]
    if norm:
        d = jnp.sqrt(jnp.sum(v * v, axis=1, keepdims=True))
    else:
        d = v[:, 0:1]
    step = (vmax - vmin) / (bins - 1)
    centers = (lax.broadcasted_iota(jnp.int32, (1, bins), 1).astype(jnp.float32)
               * step + vmin)
    gamma = 1.0 / ((vmax - vmin) / (bins - 1))
    rbf = jnp.exp(-gamma * (d - centers) ** 2)
    h = jnp.dot(rbf, w1_ref[...], preferred_element_type=jnp.float32) + b1_ref[...]
    h = _ln_silu(h, g1_ref[...], be1_ref[...])
    h = jnp.dot(h, w2_ref[...], preferred_element_type=jnp.float32) + b2_ref[...]
    o_ref[...] = _ln_silu(h, g2_ref[...], be2_ref[...])


def _embed(v, p1, p2, vmin, vmax, bins, norm):
    n, c = v.shape
    rb = _rb(n)
    d1 = p1['W'].shape[1]
    d2 = p2['W'].shape[1]
    return pl.pallas_call(
        functools.partial(_embed_body, vmin=vmin, vmax=vmax, bins=bins, norm=norm),
        grid=(n // rb,),
        in_specs=[
            pl.BlockSpec((rb, c), lambda i: (i, 0)),
            pl.BlockSpec((bins, d1), lambda i: (0, 0)),
            pl.BlockSpec((1, d1), lambda i: (0, 0)),
            pl.BlockSpec((1, d1), lambda i: (0, 0)),
            pl.BlockSpec((1, d1), lambda i: (0, 0)),
            pl.BlockSpec((d1, d2), lambda i: (0, 0)),
            pl.BlockSpec((1, d2), lambda i: (0, 0)),
            pl.BlockSpec((1, d2), lambda i: (0, 0)),
            pl.BlockSpec((1, d2), lambda i: (0, 0)),
        ],
        out_specs=pl.BlockSpec((rb, d2), lambda i: (i, 0)),
        out_shape=jax.ShapeDtypeStruct((n, d2), jnp.float32),
    )(v, p1['W'], p1['b'].reshape(1, d1), p1['g'].reshape(1, d1),
      p1['be'].reshape(1, d1), p2['W'], p2['b'].reshape(1, d2),
      p2['g'].reshape(1, d2), p2['be'].reshape(1, d2))


def _gather_body(idx_ref, t_ref, o_ref, *, rb, ch):
    c = pl.program_id(1)
    lo = c * ch

    @pl.when(c == 0)
    def _():
        o_ref[...] = jnp.zeros_like(o_ref)

    def body(i, carry):
        ix = idx_ref[0, 0, i]
        li = ix - lo

        @pl.when(jnp.logical_and(ix >= lo, ix < lo + ch))
        def _():
            o_ref[pl.ds(i, 1), :] = t_ref[pl.ds(li, 1), :]

        return carry

    lax.fori_loop(0, rb, body, 0)


def _gather(table, idx):
    n, d = table.shape
    ne = idx.shape[0]
    rb = _rb(ne)
    ch = _chunk(n, d)
    idx3 = idx.reshape(ne // rb, 1, rb)
    return pl.pallas_call(
        functools.partial(_gather_body, rb=rb, ch=ch),
        grid=(ne // rb, n // ch),
        in_specs=[
            pl.BlockSpec((1, 1, rb), lambda e, c: (e, 0, 0),
                         memory_space=pltpu.SMEM),
            pl.BlockSpec((ch, d), lambda e, c: (c, 0)),
        ],
        out_specs=pl.BlockSpec((rb, d), lambda e, c: (e, 0)),
        out_shape=jax.ShapeDtypeStruct((ne, d), jnp.float32),
    )(idx3, table)


def _scatter_body(idx_ref, v_ref, o_ref, *, rb, ch):
    e = pl.program_id(1)
    lo = pl.program_id(0) * ch

    @pl.when(e == 0)
    def _():
        o_ref[...] = jnp.zeros_like(o_ref)

    def body(i, carry):
        ix = idx_ref[0, 0, i]
        li = ix - lo

        @pl.when(jnp.logical_and(ix >= lo, ix < lo + ch))
        def _():
            o_ref[pl.ds(li, 1), :] += v_ref[pl.ds(i, 1), :]

        return carry

    lax.fori_loop(0, rb, body, 0)


def _scatter_add(vals, idx, n):
    ne, d = vals.shape
    rb = _rb(ne)
    ch = _chunk(n, d)
    idx3 = idx.reshape(ne // rb, 1, rb)
    return pl.pallas_call(
        functools.partial(_scatter_body, rb=rb, ch=ch),
        grid=(n // ch, ne // rb),
        in_specs=[
            pl.BlockSpec((1, 1, rb), lambda c, e: (e, 0, 0),
                         memory_space=pltpu.SMEM),
            pl.BlockSpec((rb, d), lambda c, e: (e, 0)),
        ],
        out_specs=pl.BlockSpec((ch, d), lambda c, e: (c, 0)),
        out_shape=jax.ShapeDtypeStruct((n, d), jnp.float32),
    )(idx3, vals)


def _msig_body(gs_ref, gd_ref, ye_ref, m_ref, cs_ref, *, h):
    gs = gs_ref[...]
    xd = gd_ref[...]
    xs = gs[:, :h]
    bh = gs[:, h:]
    m = xs + xd + ye_ref[...]
    sig = jax.nn.sigmoid(m)
    m_ref[...] = m
    cs_ref[...] = jnp.concatenate([bh * sig, sig], axis=1)


def _msig(gs, gd, ye):
    ne, h = ye.shape
    rb = _rb(ne)
    return pl.pallas_call(
        functools.partial(_msig_body, h=h),
        grid=(ne // rb,),
        in_specs=[
            pl.BlockSpec((rb, 2 * h), lambda i: (i, 0)),
            pl.BlockSpec((rb, h), lambda i: (i, 0)),
            pl.BlockSpec((rb, h), lambda i: (i, 0)),
        ],
        out_specs=[
            pl.BlockSpec((rb, h), lambda i: (i, 0)),
            pl.BlockSpec((rb, 2 * h), lambda i: (i, 0)),
        ],
        out_shape=[
            jax.ShapeDtypeStruct((ne, h), jnp.float32),
            jax.ShapeDtypeStruct((ne, 2 * h), jnp.float32),
        ],
    )(gs, gd, ye)


def _xout_body(xsu_ref, nd_ref, x_ref, g_ref, be_ref, o_ref, *, h):
    nd = nd_ref[...]
    t = xsu_ref[...] + nd[:, :h] / (nd[:, h:] + 1e-6)
    o_ref[...] = x_ref[...] + _ln_silu(t, g_ref[...], be_ref[...])


def _xout(xsu, nd, x, g, be):
    n, h = x.shape
    rb = _rb(n)
    return pl.pallas_call(
        functools.partial(_xout_body, h=h),
        grid=(n // rb,),
        in_specs=[
            pl.BlockSpec((rb, h), lambda i: (i, 0)),
            pl.BlockSpec((rb, 2 * h), lambda i: (i, 0)),
            pl.BlockSpec((rb, h), lambda i: (i, 0)),
            pl.BlockSpec((1, h), lambda i: (0, 0)),
            pl.BlockSpec((1, h), lambda i: (0, 0)),
        ],
        out_specs=pl.BlockSpec((rb, h), lambda i: (i, 0)),
        out_shape=jax.ShapeDtypeStruct((n, h), jnp.float32),
    )(xsu, nd, x, g.reshape(1, h), be.reshape(1, h))


def _yout_body(m_ref, y_ref, g_ref, be_ref, o_ref):
    o_ref[...] = y_ref[...] + _ln_silu(m_ref[...], g_ref[...], be_ref[...])


def _yout(m, y, g, be):
    n, h = y.shape
    rb = _rb(n)
    return pl.pallas_call(
        _yout_body,
        grid=(n // rb,),
        in_specs=[
            pl.BlockSpec((rb, h), lambda i: (i, 0)),
            pl.BlockSpec((rb, h), lambda i: (i, 0)),
            pl.BlockSpec((1, h), lambda i: (0, 0)),
            pl.BlockSpec((1, h), lambda i: (0, 0)),
        ],
        out_specs=pl.BlockSpec((rb, h), lambda i: (i, 0)),
        out_shape=jax.ShapeDtypeStruct((n, h), jnp.float32),
    )(m, y, g.reshape(1, h), be.reshape(1, h))


def _pool_body(x_ref, fcw_ref, fcb_ref, o_ref, acc_ref, *, n):
    i = pl.program_id(0)

    @pl.when(i == 0)
    def _():
        acc_ref[...] = jnp.zeros_like(acc_ref)

    acc_ref[0:1, :] += jnp.sum(x_ref[...], axis=0, keepdims=True)

    @pl.when(i == pl.num_programs(0) - 1)
    def _():
        o_ref[0, 0] = jnp.sum(acc_ref[0:1, :] * fcw_ref[...]) / n + fcb_ref[0, 0]


def _pool_fc(x, fcw, fcb):
    n, h = x.shape
    rb = _rb(n)
    out = pl.pallas_call(
        functools.partial(_pool_body, n=n),
        grid=(n // rb,),
        in_specs=[
            pl.BlockSpec((rb, h), lambda i: (i, 0)),
            pl.BlockSpec((1, h), lambda i: (0, 0)),
            pl.BlockSpec((1, 1), lambda i: (0, 0), memory_space=pltpu.SMEM),
        ],
        out_specs=pl.BlockSpec(memory_space=pltpu.SMEM),
        out_shape=jax.ShapeDtypeStruct((1, 1), jnp.float32),
        scratch_shapes=[pltpu.VMEM((8, h), jnp.float32)],
    )(x, fcw.reshape(1, h), fcb.reshape(1, 1))
    return out[0, 0]


def _egc(src, dst, n, x, y, p):
    h = x.shape[1]
    wcat = jnp.concatenate([p['sgW'], p['duW'], p['dgW'], p['suW']], axis=1)
    bcat = jnp.concatenate([p['sgb'], p['dub'], p['dgb'], p['sub']], axis=0)
    xcat = _dense(x, wcat, bcat)
    ye = _dense(y, p['egW'], p['egb'])
    gs = _gather(xcat[:, :2 * h], src)
    gd = _gather(xcat[:, 2 * h:3 * h], dst)
    m, cs = _msig(gs, gd, ye)
    nd = _scatter_add(cs, dst, n)
    xo = _xout(xcat[:, 3 * h:], nd, x, p['lnNg'], p['lnNb'])
    yo = _yout(m, y, p['lnEg'], p['lnEb'])
    return xo, yo


def kernel(atom_features, r, angle_h, edge_index, lg_edge_index, params):
    n_nodes = atom_features.shape[0]
    n_edges = r.shape[0]
    src, dst = edge_index[0], edge_index[1]
    lsrc, ldst = lg_edge_index[0], lg_edge_index[1]

    af = jnp.pad(atom_features, ((0, 0), (0, 128 - atom_features.shape[1])))
    aw = jnp.pad(params['atom_emb']['W'],
                 ((0, 128 - params['atom_emb']['W'].shape[0]), (0, 0)))
    x = _dense(af, aw, params['atom_emb']['b'],
               params['atom_emb']['g'], params['atom_emb']['be'])

    r8 = jnp.pad(r, ((0, 0), (0, 8 - r.shape[1])))
    y = _embed(r8, params['edge_mlp1'], params['edge_mlp2'],
               0.0, 8.0, 80, norm=True)
    z = _embed(angle_h.reshape(-1, 1), params['ang_mlp1'], params['ang_mlp2'],
               -1.0, 1.0, 40, norm=False)

    for lp in params['alignn']:
        x, m = _egc(src, dst, n_nodes, x, y, lp['node'])
        y, z = _egc(lsrc, ldst, n_edges, m, z, lp['edge'])
    for gp in params['gcn']:
        x, y = _egc(src, dst, n_nodes, x, y, gp)

    return _pool_fc(x, params['fcW'], params['fcb'])
